# baseline (speedup 1.0000x reference)
import functools

import jax
import jax.numpy as jnp
import numpy as np
from jax import lax
from jax.experimental import pallas as pl
from jax.experimental.pallas import tpu as pltpu
from jax.experimental.pallas import tpu_sc as plsc

_B, _R, _C, _D = 64, 32, 32, 768
_NROWS = _B * _R * _C
_NDROP = max(1, int(_R * _C * 0.25))
_BLK = 4096
_NW = 16
_KPW = _NDROP // _NW

_FLAT_IDS = np.array([
    61931, 49043, 33085, 42986, 33068, 45320, 22927, 21948, 18826, 2730,
    33350, 1398, 2526, 35618, 47391, 27113, 17742, 23888, 28594, 59843,
    57974, 21150, 57987, 10667, 39088, 31806, 26740, 36402, 8230, 6277,
    37093, 7978, 20085, 60437, 54307, 18134, 58541, 43685, 13055, 35051,
    56126, 26863, 34887, 16872, 8058, 18715, 46553, 35534, 65218, 17684,
    56326, 40954, 62398, 38038, 9509, 64681, 10215, 3100, 61924, 24484,
    61482, 18718, 45395, 36216, 54281, 37927, 8877, 10435, 58193, 11169,
    55617, 56316, 35851, 49713, 43823, 26752, 15385, 51995, 10666, 34560,
    53460, 1855, 31089, 38608, 20104, 1339, 21619, 12582, 22299, 14045,
    50505, 53031, 16735, 29492, 51782, 42379, 23008, 2408, 61175, 54390,
    37043, 13403, 65370, 45507, 51672, 2513, 15055, 47312, 14348, 54565,
    54190, 38368, 40512, 24535, 1673, 49500, 6593, 13079, 15740, 4259,
    45966, 20719, 13408, 31964, 63583, 36347, 19886, 65340, 55068, 3016,
    43519, 39184, 11788, 3999, 42624, 17478, 57712, 14337, 44101, 13977,
    4317, 13289, 52474, 14555, 45893, 28908, 11331, 17593, 23367, 60806,
    31018, 55797, 6615, 40573, 48479, 31587, 42176, 422, 48446, 23597,
    50241, 25844, 54417, 47226, 55817, 56495, 25618, 4794, 54490, 52919,
    2041, 49759, 42961, 9164, 53387, 2700, 36040, 31042, 44072, 773,
    61661, 17701, 36647, 38739, 1663, 38445, 42005, 63763, 59836, 163,
    34506, 4354, 20717, 63327, 16728, 12490, 28003, 22299, 5678, 50432,
    30495, 12119, 58734, 64297, 12961, 33469, 8155, 20623, 24283, 54412,
    56761, 2080, 5702, 36172, 3791, 61276, 33323, 22218, 15252, 18749,
    39046, 35143, 806, 44751, 47967, 53738, 30571, 36142, 41047, 12959,
    50435, 23661, 38582, 59202, 54037, 46662, 8101, 54440, 27748, 19659,
    31538, 57019, 62674, 52908, 57691, 28520, 39584, 54664, 63191, 56952,
    19051, 34751, 43365, 18305, 65345, 47299,
], dtype=np.int32)


def _copy_body(x_ref, o_ref):
    o_ref[...] = x_ref[...]


def _tc_copy(x2):
    return pl.pallas_call(
        _copy_body,
        grid=(_NROWS // _BLK,),
        in_specs=[pl.BlockSpec((_BLK, _D), lambda i: (i, 0))],
        out_specs=pl.BlockSpec((_BLK, _D), lambda i: (i, 0)),
        out_shape=jax.ShapeDtypeStruct((_NROWS, _D), x2.dtype),
        compiler_params=pltpu.CompilerParams(
            dimension_semantics=("arbitrary",),
        ),
    )(x2)


_mesh = plsc.VectorSubcoreMesh(
    core_axis_name="c", subcore_axis_name="s", num_cores=1, num_subcores=16
)


@functools.partial(
    pl.kernel,
    mesh=_mesh,
    scratch_types=[
        pltpu.VMEM((_KPW,), jnp.int32),
        pltpu.VMEM((_KPW, _D), jnp.float32),
        pltpu.SemaphoreType.DMA,
        pltpu.SemaphoreType.DMA,
    ],
)
def _sc_scatter(idx_hbm, tok_hbm, out_ref, idx_v, rows_v, isem, tsem):
    wid = lax.axis_index("s")
    c1 = pltpu.async_copy(idx_hbm.at[pl.ds(wid * _KPW, _KPW)], idx_v, isem)
    reps = [
        pltpu.async_copy(tok_hbm, rows_v.at[j], tsem) for j in range(_KPW)
    ]
    c1.wait()
    for r in reps:
        r.wait()
    pltpu.async_copy(rows_v, out_ref.at[idx_v], tsem).wait()


def kernel(x, mask_token):
    x2 = x.reshape(_NROWS, _D)
    idx = jnp.asarray(_FLAT_IDS)
    ref = jax.new_ref(_tc_copy(x2))
    _sc_scatter(idx, mask_token, ref)
    return jax.freeze(ref).reshape(_B, _R, _C, _D)


# speedup vs baseline: 1.0591x; 1.0591x over previous
import functools

import jax
import jax.numpy as jnp
import numpy as np
from jax import lax
from jax.experimental import pallas as pl
from jax.experimental.pallas import tpu as pltpu
from jax.experimental.pallas import tpu_sc as plsc

_B, _R, _C, _D = 64, 32, 32, 768
_NROWS = _B * _R * _C
_NDROP = max(1, int(_R * _C * 0.25))
_BLK = 4096
_NW = 16
_KPW = _NDROP // _NW

_FLAT_IDS = np.array([
    61931, 49043, 33085, 42986, 33068, 45320, 22927, 21948, 18826, 2730,
    33350, 1398, 2526, 35618, 47391, 27113, 17742, 23888, 28594, 59843,
    57974, 21150, 57987, 10667, 39088, 31806, 26740, 36402, 8230, 6277,
    37093, 7978, 20085, 60437, 54307, 18134, 58541, 43685, 13055, 35051,
    56126, 26863, 34887, 16872, 8058, 18715, 46553, 35534, 65218, 17684,
    56326, 40954, 62398, 38038, 9509, 64681, 10215, 3100, 61924, 24484,
    61482, 18718, 45395, 36216, 54281, 37927, 8877, 10435, 58193, 11169,
    55617, 56316, 35851, 49713, 43823, 26752, 15385, 51995, 10666, 34560,
    53460, 1855, 31089, 38608, 20104, 1339, 21619, 12582, 22299, 14045,
    50505, 53031, 16735, 29492, 51782, 42379, 23008, 2408, 61175, 54390,
    37043, 13403, 65370, 45507, 51672, 2513, 15055, 47312, 14348, 54565,
    54190, 38368, 40512, 24535, 1673, 49500, 6593, 13079, 15740, 4259,
    45966, 20719, 13408, 31964, 63583, 36347, 19886, 65340, 55068, 3016,
    43519, 39184, 11788, 3999, 42624, 17478, 57712, 14337, 44101, 13977,
    4317, 13289, 52474, 14555, 45893, 28908, 11331, 17593, 23367, 60806,
    31018, 55797, 6615, 40573, 48479, 31587, 42176, 422, 48446, 23597,
    50241, 25844, 54417, 47226, 55817, 56495, 25618, 4794, 54490, 52919,
    2041, 49759, 42961, 9164, 53387, 2700, 36040, 31042, 44072, 773,
    61661, 17701, 36647, 38739, 1663, 38445, 42005, 63763, 59836, 163,
    34506, 4354, 20717, 63327, 16728, 12490, 28003, 22299, 5678, 50432,
    30495, 12119, 58734, 64297, 12961, 33469, 8155, 20623, 24283, 54412,
    56761, 2080, 5702, 36172, 3791, 61276, 33323, 22218, 15252, 18749,
    39046, 35143, 806, 44751, 47967, 53738, 30571, 36142, 41047, 12959,
    50435, 23661, 38582, 59202, 54037, 46662, 8101, 54440, 27748, 19659,
    31538, 57019, 62674, 52908, 57691, 28520, 39584, 54664, 63191, 56952,
    19051, 34751, 43365, 18305, 65345, 47299,
], dtype=np.int32)


def _copy_body(x_ref, o_ref):
    o_ref[...] = x_ref[...]


def _tc_copy(x2):
    return pl.pallas_call(
        _copy_body,
        grid=(_NROWS // _BLK,),
        in_specs=[pl.BlockSpec((_BLK, _D), lambda i: (i, 0))],
        out_specs=pl.BlockSpec((_BLK, _D), lambda i: (i, 0)),
        out_shape=jax.ShapeDtypeStruct((_NROWS, _D), x2.dtype),
        compiler_params=pltpu.CompilerParams(
            dimension_semantics=("arbitrary",),
        ),
    )(x2)


_mesh = plsc.VectorSubcoreMesh(
    core_axis_name="c", subcore_axis_name="s", num_cores=1, num_subcores=16
)


@functools.partial(
    pl.kernel,
    mesh=_mesh,
    scratch_types=[
        pltpu.VMEM((_KPW,), jnp.int32),
        pltpu.VMEM((_KPW, _D), jnp.float32),
        pltpu.SemaphoreType.DMA,
        pltpu.SemaphoreType.DMA,
    ],
)
def _sc_scatter(idx_hbm, tok_hbm, out_ref, idx_v, rows_v, isem, tsem):
    wid = lax.axis_index("s")
    c1 = pltpu.async_copy(idx_hbm.at[pl.ds(wid * _KPW, _KPW)], idx_v, isem)
    c2 = pltpu.async_copy(tok_hbm, rows_v, tsem)
    c1.wait()
    c2.wait()
    pltpu.async_copy(rows_v, out_ref.at[idx_v], tsem).wait()


def kernel(x, mask_token):
    x2 = x.reshape(_NROWS, _D)
    idx = jnp.asarray(_FLAT_IDS)
    tok16 = jnp.broadcast_to(mask_token, (_KPW, _D))
    ref = jax.new_ref(_tc_copy(x2))
    _sc_scatter(idx, tok16, ref)
    return jax.freeze(ref).reshape(_B, _R, _C, _D)
